# baseline (speedup 1.0000x reference)
import functools

import jax
import jax.numpy as jnp
from jax import lax
from jax.experimental import pallas as pl
from jax.experimental.pallas import tpu as pltpu
from jax.experimental.pallas import tpu_sc as plsc

LATENT = 32
BATCH = 16384
NW = 32
B_PER_W = BATCH // NW
CHUNK = 128
NCHUNK = B_PER_W // CHUNK


def _sc_kernel(inv_hbm, stk_hbm, wi_hbm, ws_hbm, out_hbm,
               inv_idx, stk_idx, inv_rows, stk_rows, out_v, sem):
    wid = lax.axis_index("s") * 2 + lax.axis_index("c")
    base = wid * NCHUNK

    pltpu.sync_copy(inv_hbm.at[pl.ds(base, NCHUNK)], inv_idx)
    pltpu.sync_copy(stk_hbm.at[pl.ds(base, NCHUNK)], stk_idx)

    copies = []
    for j in range(NCHUNK):
        copies.append(pltpu.async_copy(wi_hbm.at[inv_idx.at[j]],
                                       inv_rows.at[pl.ds(j * CHUNK, CHUNK)],
                                       sem))
        copies.append(pltpu.async_copy(ws_hbm.at[stk_idx.at[j]],
                                       stk_rows.at[pl.ds(j * CHUNK, CHUNK)],
                                       sem))
    for c in copies:
        c.wait()

    lanes = jnp.arange(16, dtype=jnp.int32)
    for j in range(NCHUNK):
        def strip(t, _):
            rows = j * CHUNK + t * 16 + lanes
            acc = jnp.zeros((16,), jnp.float32)
            for d in range(LATENT):
                col = jnp.full((16,), d, jnp.int32)
                a = plsc.load_gather(inv_rows, [rows, col])
                b = plsc.load_gather(stk_rows, [rows, col])
                acc = acc + a * b
            out_v[j, pl.ds(t * 16, 16)] = acc
            return 0
        lax.fori_loop(0, CHUNK // 16, strip, 0)

    pltpu.sync_copy(out_v, out_hbm.at[pl.ds(base, NCHUNK)])


@functools.partial(jax.jit, static_argnames=())
def kernel(investors, stocks, W_investor, W_stock):
    mesh = plsc.VectorSubcoreMesh(core_axis_name="c", subcore_axis_name="s")
    k = functools.partial(
        pl.kernel,
        mesh=mesh,
        compiler_params=pltpu.CompilerParams(needs_layout_passes=False,
                                             use_tc_tiling_on_sc=False),
        out_type=jax.ShapeDtypeStruct((BATCH // CHUNK, CHUNK), jnp.float32),
        scratch_types=[
            pltpu.VMEM((NCHUNK, CHUNK), jnp.int32),
            pltpu.VMEM((NCHUNK, CHUNK), jnp.int32),
            pltpu.VMEM((B_PER_W, LATENT), jnp.float32),
            pltpu.VMEM((B_PER_W, LATENT), jnp.float32),
            pltpu.VMEM((NCHUNK, CHUNK), jnp.float32),
            pltpu.SemaphoreType.DMA,
        ],
    )(_sc_kernel)
    out = k(investors.reshape(BATCH // CHUNK, CHUNK),
            stocks.reshape(BATCH // CHUNK, CHUNK),
            W_investor, W_stock)
    return out.reshape(BATCH)


# speedup vs baseline: 1.0012x; 1.0012x over previous
import functools

import jax
import jax.numpy as jnp
from jax import lax
from jax.experimental import pallas as pl
from jax.experimental.pallas import tpu as pltpu
from jax.experimental.pallas import tpu_sc as plsc

LATENT = 32
BATCH = 16384
NW = 32
B_PER_W = BATCH // NW
CHUNK = 128
NCHUNK = B_PER_W // CHUNK


def _fire_chunk(wi_hbm, ws_hbm, inv_idx, stk_idx, inv_rows, stk_rows, j,
                slot, sem):
    return [
        pltpu.async_copy(wi_hbm.at[inv_idx.at[j]],
                         inv_rows.at[slot], sem),
        pltpu.async_copy(ws_hbm.at[stk_idx.at[j]],
                         stk_rows.at[slot], sem),
    ]


def _sc_kernel(inv_hbm, stk_hbm, wi_hbm, ws_hbm, out_hbm,
               inv_idx, stk_idx, inv_rows, stk_rows, out_v, sems):
    wid = lax.axis_index("s") * 2 + lax.axis_index("c")
    base = wid * NCHUNK

    pltpu.sync_copy(inv_hbm.at[pl.ds(base, NCHUNK)], inv_idx)
    pltpu.sync_copy(stk_hbm.at[pl.ds(base, NCHUNK)], stk_idx)

    pending = _fire_chunk(wi_hbm, ws_hbm, inv_idx, stk_idx,
                          inv_rows, stk_rows, 0, 0, sems.at[0])
    lanes = jnp.arange(16, dtype=jnp.int32)
    for j in range(NCHUNK):
        slot = j % 2
        if j + 1 < NCHUNK:
            nxt = _fire_chunk(wi_hbm, ws_hbm, inv_idx, stk_idx,
                              inv_rows, stk_rows, j + 1, (j + 1) % 2,
                              sems.at[(j + 1) % 2])
        else:
            nxt = []
        for c in pending:
            c.wait()
        pending = nxt

        def strip(t, _):
            rows = t * 16 + lanes
            acc = jnp.zeros((16,), jnp.float32)
            for d in range(LATENT):
                col = jnp.full((16,), d, jnp.int32)
                a = plsc.load_gather(inv_rows.at[slot], [rows, col])
                b = plsc.load_gather(stk_rows.at[slot], [rows, col])
                acc = acc + a * b
            out_v[j, pl.ds(t * 16, 16)] = acc
            return 0
        lax.fori_loop(0, CHUNK // 16, strip, 0)

    pltpu.sync_copy(out_v, out_hbm.at[pl.ds(base, NCHUNK)])


@jax.jit
def kernel(investors, stocks, W_investor, W_stock):
    mesh = plsc.VectorSubcoreMesh(core_axis_name="c", subcore_axis_name="s")
    k = functools.partial(
        pl.kernel,
        mesh=mesh,
        compiler_params=pltpu.CompilerParams(needs_layout_passes=False,
                                             use_tc_tiling_on_sc=False),
        out_type=jax.ShapeDtypeStruct((BATCH // CHUNK, CHUNK), jnp.float32),
        scratch_types=[
            pltpu.VMEM((NCHUNK, CHUNK), jnp.int32),
            pltpu.VMEM((NCHUNK, CHUNK), jnp.int32),
            pltpu.VMEM((2, CHUNK, LATENT), jnp.float32),
            pltpu.VMEM((2, CHUNK, LATENT), jnp.float32),
            pltpu.VMEM((NCHUNK, CHUNK), jnp.float32),
            pltpu.SemaphoreType.DMA((2,)),
        ],
    )(_sc_kernel)
    out = k(investors.reshape(BATCH // CHUNK, CHUNK),
            stocks.reshape(BATCH // CHUNK, CHUNK),
            W_investor, W_stock)
    return out.reshape(BATCH)
